# baseline (speedup 1.0000x reference)
import functools

import jax
import jax.numpy as jnp
from jax import lax
from jax.experimental import pallas as pl
from jax.experimental.pallas import tpu as pltpu
from jax.experimental.pallas import tpu_sc as plsc

B, F = 16384, 100
N = B * F
NC, NS, L = 2, 16, 16
NW = NC * NS
PW = N // NW
CH = 12800
NCH = PW // CH

ONE_BITS = 0x3F800000


@functools.partial(
    pl.kernel,
    mesh=plsc.VectorSubcoreMesh(core_axis_name="c", subcore_axis_name="s"),
    out_type=jax.ShapeDtypeStruct((2 * N,), jnp.float32),
    scratch_types=[
        pltpu.VMEM((CH,), jnp.int32),
        pltpu.VMEM((2 * CH,), jnp.float32),
    ],
    compiler_params=pltpu.CompilerParams(needs_layout_passes=False),
)
def _onehot_sc(x_hbm, out_hbm, xin, oout):
    wid = lax.axis_index("s") * NC + lax.axis_index("c")
    base = wid * PW
    even = 2 * lax.iota(jnp.int32, L)

    def chunk_body(c, carry):
        off = base + c * CH
        pltpu.sync_copy(x_hbm.at[pl.ds(off, CH)], xin)

        def body(j, carry2):
            x = xin[pl.ds(j * L, L)]
            v = x.astype(jnp.float32)
            u = 1.0 - v
            idx = even + (2 * L) * j
            plsc.store_scatter(oout, [idx], u)
            plsc.store_scatter(oout, [idx + 1], v)
            return carry2

        lax.fori_loop(0, CH // L, body, 0, unroll=4)
        pltpu.sync_copy(oout, out_hbm.at[pl.ds(2 * off, 2 * CH)])
        return carry

    lax.fori_loop(0, NCH, chunk_body, 0)


def kernel(inputs):
    x = inputs.astype(jnp.int32).reshape(N)
    return _onehot_sc(x).reshape(B, F, 2)


# speedup vs baseline: 17.9790x; 17.9790x over previous
import functools

import jax
import jax.numpy as jnp
from jax.experimental import pallas as pl
from jax.experimental.pallas import tpu as pltpu

B, F = 16384, 100
FB = 8
BB = 2048
GRID_F = (F + FB - 1) // FB
GRID_B = B // BB


def _block(x_ref, o_ref):
    v = x_ref[...].astype(jnp.float32)
    v3 = v.reshape(FB, BB // 128, 128)
    u3 = 1.0 - v3
    o_ref[...] = jnp.stack([u3, v3], axis=2).reshape(FB, 2 * (BB // 128), 128)


_onehot = pl.pallas_call(
    _block,
    grid=(GRID_F, GRID_B),
    in_specs=[pl.BlockSpec((FB, BB), lambda i, j: (i, j))],
    out_specs=pl.BlockSpec((FB, 2 * (BB // 128), 128), lambda i, j: (i, j, 0)),
    out_shape=jax.ShapeDtypeStruct((F, 2 * (B // 128), 128), jnp.float32),
)


def kernel(inputs):
    xt = inputs.astype(jnp.int32).T
    o3 = _onehot(xt)
    o4 = o3.reshape(F, B // 128, 2, 128)
    return o4.transpose(1, 3, 0, 2).reshape(B, F, 2)


# speedup vs baseline: 19.2776x; 1.0722x over previous
import functools

import jax
import jax.numpy as jnp
from jax.experimental import pallas as pl
from jax.experimental.pallas import tpu as pltpu

B, F = 16384, 100
FB = 8
BB = 2048
GRID_F = (F + FB - 1) // FB
GRID_B = B // BB


def _block(x_ref, o_ref):
    v = x_ref[...].astype(jnp.float32)
    v3 = v.reshape(FB, BB // 128, 128)
    u3 = 1.0 - v3
    o_ref[:, ::2, :] = u3
    o_ref[:, 1::2, :] = v3


_onehot = pl.pallas_call(
    _block,
    grid=(GRID_F, GRID_B),
    in_specs=[pl.BlockSpec((FB, BB), lambda i, j: (i, j))],
    out_specs=pl.BlockSpec((FB, 2 * (BB // 128), 128), lambda i, j: (i, j, 0)),
    out_shape=jax.ShapeDtypeStruct((F, 2 * (B // 128), 128), jnp.float32),
)


def kernel(inputs):
    xt = inputs.astype(jnp.int32).T
    o3 = _onehot(xt)
    o4 = o3.reshape(F, B // 128, 2, 128)
    return o4.transpose(1, 3, 0, 2).reshape(B, F, 2)


# speedup vs baseline: 86.1952x; 4.4713x over previous
import functools

import jax
import jax.numpy as jnp
from jax.experimental import pallas as pl
from jax.experimental.pallas import tpu as pltpu

B, F = 16384, 100
FB = 8
BB = 16384
GRID_F = (F + FB - 1) // FB
GRID_B = B // BB


def _block(x_ref, o_ref):
    v = x_ref[...].astype(jnp.float32)
    v3 = v.reshape(FB, BB // 128, 128)
    u3 = 1.0 - v3
    o_ref[:, ::2, :] = u3
    o_ref[:, 1::2, :] = v3


_onehot = pl.pallas_call(
    _block,
    grid=(GRID_F, GRID_B),
    in_specs=[pl.BlockSpec((FB, BB), lambda i, j: (i, j))],
    out_specs=pl.BlockSpec((FB, 2 * (BB // 128), 128), lambda i, j: (i, j, 0)),
    out_shape=jax.ShapeDtypeStruct((F, 2 * (B // 128), 128), jnp.float32),
)


def kernel(inputs):
    xt = inputs.astype(jnp.int32).T
    o3 = _onehot(xt)
    o4 = o3.reshape(F, B // 128, 2, 128)
    return o4.transpose(1, 3, 0, 2).reshape(B, F, 2)


# speedup vs baseline: 110.1855x; 1.2783x over previous
import functools

import jax
import jax.numpy as jnp
from jax.experimental import pallas as pl
from jax.experimental.pallas import tpu as pltpu

B, F = 16384, 100
FB = 16
BB = 16384
GRID_F = (F + FB - 1) // FB
GRID_B = B // BB


def _block(x_ref, o_ref):
    v = x_ref[...].astype(jnp.float32)
    v3 = v.reshape(FB, BB // 128, 128)
    u3 = 1.0 - v3
    o_ref[:, ::2, :] = u3
    o_ref[:, 1::2, :] = v3


_onehot = pl.pallas_call(
    _block,
    grid=(GRID_F, GRID_B),
    in_specs=[pl.BlockSpec((FB, BB), lambda i, j: (i, j))],
    out_specs=pl.BlockSpec((FB, 2 * (BB // 128), 128), lambda i, j: (i, j, 0)),
    out_shape=jax.ShapeDtypeStruct((F, 2 * (B // 128), 128), jnp.float32),
)


def kernel(inputs):
    xt = inputs.astype(jnp.int32).T
    o3 = _onehot(xt)
    o4 = o3.reshape(F, B // 128, 2, 128)
    return o4.transpose(1, 3, 0, 2).reshape(B, F, 2)


# speedup vs baseline: 131.6195x; 1.1945x over previous
import functools

import jax
import jax.numpy as jnp
from jax.experimental import pallas as pl
from jax.experimental.pallas import tpu as pltpu

B, F = 16384, 100
FB = 32
BB = 16384
GRID_F = (F + FB - 1) // FB
GRID_B = B // BB


def _block(x_ref, o_ref):
    v = x_ref[...].astype(jnp.float32)
    v3 = v.reshape(FB, BB // 128, 128)
    u3 = 1.0 - v3
    o_ref[:, ::2, :] = u3
    o_ref[:, 1::2, :] = v3


_onehot = pl.pallas_call(
    _block,
    grid=(GRID_F, GRID_B),
    in_specs=[pl.BlockSpec((FB, BB), lambda i, j: (i, j))],
    out_specs=pl.BlockSpec((FB, 2 * (BB // 128), 128), lambda i, j: (i, j, 0)),
    out_shape=jax.ShapeDtypeStruct((F, 2 * (B // 128), 128), jnp.float32),
)


def kernel(inputs):
    xt = inputs.astype(jnp.int32).T
    o3 = _onehot(xt)
    o4 = o3.reshape(F, B // 128, 2, 128)
    return o4.transpose(1, 3, 0, 2).reshape(B, F, 2)


# speedup vs baseline: 134.8115x; 1.0243x over previous
import functools

import jax
import jax.numpy as jnp
from jax.experimental import pallas as pl
from jax.experimental.pallas import tpu as pltpu

B, F = 16384, 100
FB = 56
BB = 16384
GRID_F = (F + FB - 1) // FB
GRID_B = B // BB


def _block(x_ref, o_ref):
    v = x_ref[...].astype(jnp.float32)
    v3 = v.reshape(FB, BB // 128, 128)
    u3 = 1.0 - v3
    o_ref[:, ::2, :] = u3
    o_ref[:, 1::2, :] = v3


_onehot = pl.pallas_call(
    _block,
    grid=(GRID_F, GRID_B),
    in_specs=[pl.BlockSpec((FB, BB), lambda i, j: (i, j))],
    out_specs=pl.BlockSpec((FB, 2 * (BB // 128), 128), lambda i, j: (i, j, 0)),
    out_shape=jax.ShapeDtypeStruct((F, 2 * (B // 128), 128), jnp.float32),
)


def kernel(inputs):
    xt = inputs.astype(jnp.int32).T
    o3 = _onehot(xt)
    o4 = o3.reshape(F, B // 128, 2, 128)
    return o4.transpose(1, 3, 0, 2).reshape(B, F, 2)


# speedup vs baseline: 138.4988x; 1.0274x over previous
import functools

import jax
import jax.numpy as jnp
from jax.experimental import pallas as pl
from jax.experimental.pallas import tpu as pltpu

B, F = 16384, 100
FB = 40
BB = 16384
GRID_F = (F + FB - 1) // FB
GRID_B = B // BB


def _block(x_ref, o_ref):
    v = x_ref[...].astype(jnp.float32)
    v3 = v.reshape(FB, BB // 128, 128)
    u3 = 1.0 - v3
    o_ref[:, ::2, :] = u3
    o_ref[:, 1::2, :] = v3


_onehot = pl.pallas_call(
    _block,
    grid=(GRID_F, GRID_B),
    in_specs=[pl.BlockSpec((FB, BB), lambda i, j: (i, j))],
    out_specs=pl.BlockSpec((FB, 2 * (BB // 128), 128), lambda i, j: (i, j, 0)),
    out_shape=jax.ShapeDtypeStruct((F, 2 * (B // 128), 128), jnp.float32),
)


def kernel(inputs):
    xt = inputs.astype(jnp.int32).T
    o3 = _onehot(xt)
    o4 = o3.reshape(F, B // 128, 2, 128)
    return o4.transpose(1, 3, 0, 2).reshape(B, F, 2)
